# baseline (speedup 1.0000x reference)
import jax
import jax.numpy as jnp
from jax import lax
from jax.experimental import pallas as pl
from jax.experimental.pallas import tpu as pltpu
from jax.experimental.pallas import tpu_sc as plsc

_B = 8
_N = 100000
_NPB = 102400
_CPB = 12800
_H = 256
_W = 256
_TCOLS = 3200
_CH = 640
_NCH = _TCOLS // _CH
_NSLOT = 4
_NGRP = _CH // 16
_LIM3 = (40, 10, 0, 0, 0)

_MAGIC = 8388608.0


def _sc_body(xy_hbm, out_hbm, buf_v, grid_v, shared_s, sem0, sem1, sem2, sem3):
    s = lax.axis_index("s")
    c = lax.axis_index("c")
    lb = s // 4
    q = s % 4
    b = c * 4 + lb
    colbase = q * _TCOLS
    sems = (sem0, sem1, sem2, sem3)

    zeros16 = jnp.zeros((16,), jnp.float32)
    ones16 = jnp.ones((16,), jnp.float32)

    def chunk_copy(ci, slot):
        return pltpu.make_async_copy(
            xy_hbm.at[pl.ds(b * 16, 16), pl.ds(colbase + ci * _CH, _CH)],
            buf_v.at[slot, pl.ds(0, 16), pl.ds(0, _CH)], sems[slot])

    for k in range(_NSLOT):
        chunk_copy(k, k).start()

    def zrow(r, carry):
        for k in range(_W // 16):
            grid_v[r, pl.ds(k * 16, 16)] = zeros16
        return carry

    lax.fori_loop(0, _H, zrow, 0)

    def do_group(slot, r, off):
        xv = buf_v[slot, r, pl.ds(off, 16)]
        yv = buf_v[slot, 8 + r, pl.ds(off, 16)]
        rx = (xv * 255.0 + _MAGIC) - _MAGIC
        ry = (yv * 255.0 + _MAGIC) - _MAGIC
        ix = rx.astype(jnp.int32)
        iy = ry.astype(jnp.int32)
        plsc.store_scatter(grid_v, [ix, iy], ones16)

    for ci in range(_NCH):
        slot = ci % _NSLOT
        chunk_copy(ci, slot).wait()

        for r in range(7):
            @plsc.parallel_loop(0, _NGRP, unroll=4)
            def _grp(g, slot=slot, r=r):
                do_group(slot, r, g * 16)

        r7bound = jnp.where(q == 3, _LIM3[ci], _NGRP)

        @plsc.parallel_loop(0, r7bound, unroll=2)
        def _grp7(g, slot=slot):
            do_group(slot, 7, g * 16)

        if ci + _NSLOT < _NCH:
            chunk_copy(ci + _NSLOT, slot).start()

    for w in range(1, 4):
        jw = lax.rem(q + w, 4)
        pltpu.sync_copy(grid_v.at[pl.ds(jw * 64, 64)],
                        shared_s.at[pl.ds((lb * 4 + jw) * 64, 64)])
        plsc.subcore_barrier()
        pltpu.sync_copy(shared_s.at[pl.ds((lb * 4 + q) * 64, 64)],
                        grid_v.at[pl.ds(jw * 64, 64)])
        plsc.subcore_barrier()

    @plsc.parallel_loop(0, 64, unroll=2)
    def _acc(r):
        for k in range(_W // 16):
            cs = pl.ds(k * 16, 16)
            v = (grid_v[r, cs] + grid_v[64 + r, cs]
                 + grid_v[128 + r, cs] + grid_v[192 + r, cs])
            grid_v[r, cs] = jnp.minimum(v, 1.0)

    pltpu.sync_copy(grid_v.at[pl.ds(0, 64)], out_hbm.at[b, pl.ds(q * 64, 64)])


def kernel(s_cur, action):
    del action
    t = jnp.pad(s_cur[:, :, 0:2], ((0, 0), (0, _NPB - _N), (0, 0)))
    xy = (t.reshape(_B, 8, _CPB, 2).transpose(0, 3, 1, 2)
          .reshape(_B * 16, _CPB))
    mesh = plsc.VectorSubcoreMesh(core_axis_name="c", subcore_axis_name="s")
    occ = pl.kernel(
        _sc_body,
        mesh=mesh,
        compiler_params=pltpu.CompilerParams(needs_layout_passes=False),
        out_type=jax.ShapeDtypeStruct((_B, _H, _W), jnp.float32),
        scratch_types=[
            pltpu.VMEM((_NSLOT, 16, _CH), jnp.float32),
            pltpu.VMEM((_H, _W), jnp.float32),
            pltpu.VMEM_SHARED((1024, _W), jnp.float32),
            pltpu.SemaphoreType.DMA,
            pltpu.SemaphoreType.DMA,
            pltpu.SemaphoreType.DMA,
            pltpu.SemaphoreType.DMA,
        ],
    )(xy)
    return occ


# speedup vs baseline: 1.2455x; 1.2455x over previous
import jax
import jax.numpy as jnp
from jax import lax
from jax.experimental import pallas as pl
from jax.experimental.pallas import tpu as pltpu
from jax.experimental.pallas import tpu_sc as plsc

_B = 8
_N = 100000
_NPB = 102400
_CPB = 12800
_H = 256
_W = 256
_TCOLS = 3200
_CH = 640
_NCH = _TCOLS // _CH
_NSLOT = 4
_NGRP = _CH // 16
_LIM3 = (40, 10, 0, 0, 0)

_MAGIC = 8388608.0


def _sc_body(xy_hbm, out_hbm, buf_v, grid_v, shared_s, sem0, sem1, sem2, sem3):
    s = lax.axis_index("s")
    c = lax.axis_index("c")
    lb = s // 4
    q = s % 4
    b = c * 4 + lb
    colbase = q * _TCOLS
    sems = (sem0, sem1, sem2, sem3)

    zeros16 = jnp.zeros((16,), jnp.float32)
    ones16 = jnp.ones((16,), jnp.float32)

    def chunk_copy(ci, slot):
        return pltpu.make_async_copy(
            xy_hbm.at[pl.ds(b * 16, 16), pl.ds(colbase + ci * _CH, _CH)],
            buf_v.at[slot, pl.ds(0, 16), pl.ds(0, _CH)], sems[slot])

    for k in range(_NSLOT):
        chunk_copy(k, k).start()

    def zrow(r, carry):
        for k in range(_W // 16):
            grid_v[r, pl.ds(k * 16, 16)] = zeros16
        return carry

    lax.fori_loop(0, _H, zrow, 0)

    def do_group(slot, r, off):
        xv = buf_v[slot, r, pl.ds(off, 16)]
        yv = buf_v[slot, 8 + r, pl.ds(off, 16)]
        rx = (xv * 255.0 + _MAGIC) - _MAGIC
        ry = (yv * 255.0 + _MAGIC) - _MAGIC
        ix = rx.astype(jnp.int32)
        iy = ry.astype(jnp.int32)
        plsc.store_scatter(grid_v, [ix, iy], ones16)

    for ci in range(_NCH):
        slot = ci % _NSLOT
        chunk_copy(ci, slot).wait()

        for r in range(7):
            @plsc.parallel_loop(0, _NGRP, unroll=4)
            def _grp(g, slot=slot, r=r):
                do_group(slot, r, g * 16)

        r7bound = jnp.where(q == 3, _LIM3[ci], _NGRP)

        @plsc.parallel_loop(0, r7bound, unroll=2)
        def _grp7(g, slot=slot):
            do_group(slot, 7, g * 16)

        if ci + _NSLOT < _NCH:
            chunk_copy(ci + _NSLOT, slot).start()

    for w in range(1, 4):
        jw = lax.rem(q + w, 4)
        pltpu.sync_copy(grid_v.at[pl.ds(jw * 64, 64)],
                        shared_s.at[pl.ds((lb * 4 + jw) * 64, 64)])
        plsc.subcore_barrier()
        pltpu.sync_copy(shared_s.at[pl.ds((lb * 4 + q) * 64, 64)],
                        grid_v.at[pl.ds(jw * 64, 64)])
        plsc.subcore_barrier()

    @plsc.parallel_loop(0, 64, unroll=2)
    def _acc(r):
        for k in range(_W // 16):
            cs = pl.ds(k * 16, 16)
            v = (grid_v[r, cs] + grid_v[64 + r, cs]
                 + grid_v[128 + r, cs] + grid_v[192 + r, cs])
            grid_v[r, cs] = jnp.minimum(v, 1.0)

    pltpu.sync_copy(grid_v.at[pl.ds(0, 64)], out_hbm.at[b, pl.ds(q * 64, 64)])


def kernel(s_cur, action):
    del action
    xs, ys = s_cur[:, :, 0], s_cur[:, :, 1]
    xp = jnp.pad(xs, ((0, 0), (0, _NPB - _N))).reshape(_B, 8, _CPB)
    yp = jnp.pad(ys, ((0, 0), (0, _NPB - _N))).reshape(_B, 8, _CPB)
    xy = jnp.concatenate([xp, yp], axis=1).reshape(_B * 16, _CPB)
    mesh = plsc.VectorSubcoreMesh(core_axis_name="c", subcore_axis_name="s")
    occ = pl.kernel(
        _sc_body,
        mesh=mesh,
        compiler_params=pltpu.CompilerParams(needs_layout_passes=False),
        out_type=jax.ShapeDtypeStruct((_B, _H, _W), jnp.float32),
        scratch_types=[
            pltpu.VMEM((_NSLOT, 16, _CH), jnp.float32),
            pltpu.VMEM((_H, _W), jnp.float32),
            pltpu.VMEM_SHARED((1024, _W), jnp.float32),
            pltpu.SemaphoreType.DMA,
            pltpu.SemaphoreType.DMA,
            pltpu.SemaphoreType.DMA,
            pltpu.SemaphoreType.DMA,
        ],
    )(xy)
    return occ
